# baseline (speedup 1.0000x reference)
import functools

import jax
import jax.numpy as jnp
from jax import lax
from jax.experimental import pallas as pl
from jax.experimental.pallas import tpu as pltpu
from jax.experimental.pallas import tpu_sc as plsc

_INTERP = False

_NC = 2
_NS = 16
_NW = _NC * _NS


def _sc_gather(table, idx_flat):
    V, D = table.shape
    dt = table.dtype
    E = idx_flat.shape[0]
    assert E % _NW == 0 and D % 128 == 0
    b_per_w = E // _NW
    CH = 200
    assert b_per_w % (2 * CH) == 0 and CH % 8 == 0
    n_pairs = b_per_w // (2 * CH)

    mesh = plsc.VectorSubcoreMesh(core_axis_name="c", subcore_axis_name="s",
                                  num_cores=_NC)

    @functools.partial(
        pl.kernel, mesh=mesh,
        out_type=jax.ShapeDtypeStruct((E, D), dt),
        scratch_types=[
            pltpu.VMEM((CH,), jnp.int32),
            pltpu.VMEM((CH,), jnp.int32),
            pltpu.VMEM((CH, D), dt),
            pltpu.VMEM((CH, D), dt),
            pltpu.SemaphoreType.DMA,
            pltpu.SemaphoreType.DMA,
            pltpu.SemaphoreType.DMA,
            pltpu.SemaphoreType.DMA,
        ],
    )
    def k(table_hbm, idx_hbm, out_hbm, idx_a, idx_b, rows_a, rows_b,
          gsem_a, gsem_b, wsem_a, wsem_b):
        wid = lax.axis_index("s") * _NC + lax.axis_index("c")
        base = wid * b_per_w

        def start_gather(off, idx_v, rows_v, gsem):
            pltpu.sync_copy(idx_hbm.at[pl.ds(off, CH)], idx_v)
            pltpu.async_copy(table_hbm.at[idx_v], rows_v, gsem)

        def wait_gather(idx_v, rows_v, gsem):
            pltpu.make_async_copy(table_hbm.at[idx_v], rows_v, gsem).wait()

        def drain_write(rows_v, wsem):
            pltpu.make_async_copy(rows_v, out_hbm.at[pl.ds(base, CH)],
                                  wsem).wait()

        start_gather(base, idx_a, rows_a, gsem_a)

        def body(j, carry):
            off0 = base + (2 * j) * CH
            off1 = off0 + CH

            wait_gather(idx_a, rows_a, gsem_a)
            pltpu.async_copy(rows_a, out_hbm.at[pl.ds(off0, CH)], wsem_a)

            @pl.when(j > 0)
            def _():
                drain_write(rows_b, wsem_b)

            start_gather(off1, idx_b, rows_b, gsem_b)

            wait_gather(idx_b, rows_b, gsem_b)
            pltpu.async_copy(rows_b, out_hbm.at[pl.ds(off1, CH)], wsem_b)
            drain_write(rows_a, wsem_a)

            @pl.when(j + 1 < n_pairs)
            def _():
                start_gather(off1 + CH, idx_a, rows_a, gsem_a)

            return carry

        lax.fori_loop(0, n_pairs, body, 0)
        drain_write(rows_b, wsem_b)

    return k(table, idx_flat)


def _conv_call(x, gath, nbrf, Ws, Wf, PA, PB, Wnext, y_dtype):
    N, AF = x.shape
    E, NF = nbrf.shape
    C = gath.shape[1]
    M = E // N
    A = 200
    T = N // A
    ET = A * M
    cnt1 = float(E)

    def body(x_ref, g_ref, nb_ref, ws_ref, wf_ref, pa_ref, pb_ref, wn_ref,
             xo_ref, yo_ref, ns_ref, st_ref, s2_ref):
        p = pl.program_id(0)
        t = pl.program_id(1)

        @pl.when(jnp.logical_and(p == 0, t == 0))
        def _init():
            st_ref[...] = jnp.zeros_like(st_ref)

        def gated():
            x_t = x_ref[pl.ds(t * A, A), :]
            s2 = jnp.dot(x_t, ws_ref[...],
                         preferred_element_type=jnp.float32) + pa_ref[0:1, :]
            f2 = jnp.dot(nb_ref[...], wf_ref[...],
                         preferred_element_type=jnp.float32)
            g = g_ref[...].astype(jnp.float32)
            z = (g + f2).reshape(A, M, C) + s2[:, None, :]
            return z

        @pl.when(p == 0)
        def _pass0():
            z = gated()
            st_ref[0:1, :] += jnp.sum(z, axis=(0, 1))[None, :]
            st_ref[1:2, :] += jnp.sum(z * z, axis=(0, 1))[None, :]

        @pl.when(jnp.logical_and(p == 1, t == 0))
        def _mkstats():
            m = st_ref[0:1, :] / cnt1
            v = st_ref[1:2, :] / cnt1 - m * m
            sc = pa_ref[1:2, :] * lax.rsqrt(v + 1e-5)
            st_ref[2:3, :] = sc
            st_ref[3:4, :] = pa_ref[2:3, :] - m * sc

        @pl.when(p == 1)
        def _pass1():
            z = gated()
            zn = z * st_ref[2:3, :][:, None, :] + st_ref[3:4, :][:, None, :]
            filt = jax.nn.sigmoid(zn[:, :, :AF])
            core = jax.nn.softplus(zn[:, :, AF:])
            ns_ref[pl.ds(t * A, A), :] = jnp.sum(filt * core, axis=1)

        @pl.when(jnp.logical_and(p == 2, t == 0))
        def _mkstats2():
            ns = ns_ref[...]
            m2 = jnp.mean(ns, axis=0, keepdims=True)
            v2 = jnp.mean((ns - m2) ** 2, axis=0, keepdims=True)
            sc2 = pb_ref[0:1, :] * lax.rsqrt(v2 + 1e-5)
            s2_ref[0:1, :] = sc2
            s2_ref[1:2, :] = pb_ref[1:2, :] - m2 * sc2

        @pl.when(p == 2)
        def _pass2():
            x_t = x_ref[pl.ds(t * A, A), :]
            ns = ns_ref[pl.ds(t * A, A), :]
            xn = jax.nn.softplus(x_t + ns * s2_ref[0:1, :] + s2_ref[1:2, :])
            xo_ref[...] = xn
            yo_ref[...] = jnp.dot(xn, wn_ref[...],
                                  preferred_element_type=jnp.float32
                                  ).astype(yo_ref.dtype)

    grid = (3, T)
    last = T - 1

    def edge_map(p, t):
        return (jnp.where(p == 2, last, t), 0)

    return pl.pallas_call(
        body,
        grid=grid,
        in_specs=[
            pl.BlockSpec((N, AF), lambda p, t: (0, 0)),
            pl.BlockSpec((ET, C), edge_map),
            pl.BlockSpec((ET, NF), edge_map),
            pl.BlockSpec((AF, C), lambda p, t: (0, 0)),
            pl.BlockSpec((NF, C), lambda p, t: (0, 0)),
            pl.BlockSpec((3, C), lambda p, t: (0, 0)),
            pl.BlockSpec((2, AF), lambda p, t: (0, 0)),
            pl.BlockSpec((AF, C), lambda p, t: (0, 0)),
        ],
        out_specs=[
            pl.BlockSpec((A, AF), lambda p, t: (t, 0)),
            pl.BlockSpec((A, C), lambda p, t: (t, 0)),
        ],
        out_shape=[
            jax.ShapeDtypeStruct((N, AF), jnp.float32),
            jax.ShapeDtypeStruct((N, C), y_dtype),
        ],
        scratch_shapes=[
            pltpu.VMEM((N, AF), jnp.float32),
            pltpu.VMEM((4, C), jnp.float32),
            pltpu.VMEM((2, AF), jnp.float32),
        ],
        interpret=_INTERP,
    )(x, gath, nbrf, Ws, Wf, PA, PB, Wnext)


def _emb_call(atom_fea, W_emb, b_emb2, Wn0):
    N, ORIG = atom_fea.shape
    AF = W_emb.shape[1]
    C = Wn0.shape[1]
    R = 1000
    T = N // R

    def body(a_ref, w_ref, b_ref, wn_ref, x_ref, y_ref):
        x = jnp.dot(a_ref[...], w_ref[...],
                    preferred_element_type=jnp.float32) + b_ref[...]
        x_ref[...] = x
        y_ref[...] = jnp.dot(x, wn_ref[...],
                             preferred_element_type=jnp.float32
                             ).astype(y_ref.dtype)

    return pl.pallas_call(
        body,
        grid=(T,),
        in_specs=[
            pl.BlockSpec((R, ORIG), lambda t: (t, 0)),
            pl.BlockSpec((ORIG, AF), lambda t: (0, 0)),
            pl.BlockSpec((1, AF), lambda t: (0, 0)),
            pl.BlockSpec((AF, C), lambda t: (0, 0)),
        ],
        out_specs=[
            pl.BlockSpec((R, AF), lambda t: (t, 0)),
            pl.BlockSpec((R, C), lambda t: (t, 0)),
        ],
        out_shape=[
            jax.ShapeDtypeStruct((N, AF), jnp.float32),
            jax.ShapeDtypeStruct((N, C), jnp.float32),
        ],
        interpret=_INTERP,
    )(atom_fea, W_emb, b_emb2, Wn0)


def _head_call(y3, b_fc2, W_out_row, b_out2, B, PER):
    N, H = y3.shape

    def body(y_ref, bfc_ref, wo_ref, bo_ref, out_ref, cf_ref):
        crys = jnp.mean(y_ref[...].reshape(B, PER, H), axis=1)
        cf = jax.nn.softplus(crys + bfc_ref[...])
        cf_ref[...] = cf
        out_ref[...] = (jnp.sum(cf * wo_ref[...], axis=1, keepdims=True)
                        + bo_ref[...])

    return pl.pallas_call(
        body,
        grid=(1,),
        in_specs=[
            pl.BlockSpec((N, H), lambda i: (0, 0)),
            pl.BlockSpec((1, H), lambda i: (0, 0)),
            pl.BlockSpec((1, H), lambda i: (0, 0)),
            pl.BlockSpec((1, 1), lambda i: (0, 0)),
        ],
        out_specs=[
            pl.BlockSpec((B, 1), lambda i: (0, 0)),
            pl.BlockSpec((B, H), lambda i: (0, 0)),
        ],
        out_shape=[
            jax.ShapeDtypeStruct((B, 1), jnp.float32),
            jax.ShapeDtypeStruct((B, H), jnp.float32),
        ],
        interpret=_INTERP,
    )(y3, b_fc2, W_out_row, b_out2)


def kernel(atom_fea, nbr_fea, nbr_fea_idx, crystal_atom_idx,
           W_emb, b_emb,
           W_full0, b_full0, bn1_g0, bn1_b0, bn2_g0, bn2_b0,
           W_full1, b_full1, bn1_g1, bn1_b1, bn2_g1, bn2_b1,
           W_full2, b_full2, bn1_g2, bn1_b2, bn2_g2, bn2_b2,
           W_fc, b_fc, W_out, b_out):
    N, M, NF = nbr_fea.shape
    AF = W_emb.shape[1]
    B, PER = crystal_atom_idx.shape

    idx_flat = nbr_fea_idx.reshape(N * M).astype(jnp.int32)
    nbrf = nbr_fea.reshape(N * M, NF)

    layers = [
        (W_full0, b_full0, bn1_g0, bn1_b0, bn2_g0, bn2_b0),
        (W_full1, b_full1, bn1_g1, bn1_b1, bn2_g1, bn2_b1),
        (W_full2, b_full2, bn1_g2, bn1_b2, bn2_g2, bn2_b2),
    ]

    x, y = _emb_call(atom_fea, W_emb, b_emb.reshape(1, AF),
                     layers[0][0][AF:2 * AF, :])

    for li, (W_full, b_full, g1, be1, g2, be2) in enumerate(layers):
        Ws = W_full[:AF, :]
        Wf = W_full[2 * AF:, :]
        PA = jnp.stack([b_full, g1, be1], axis=0)
        PB = jnp.stack([g2, be2], axis=0)
        Wnext = layers[li + 1][0][AF:2 * AF, :] if li < 2 else W_fc
        gath = _sc_gather(y, idx_flat)
        x, y = _conv_call(x, gath, nbrf, Ws, Wf, PA, PB, Wnext, jnp.float32)

    out, crys_fea = _head_call(y, b_fc.reshape(1, -1),
                               W_out.reshape(1, -1), b_out.reshape(1, 1),
                               B, PER)
    return (out, crys_fea)


# speedup vs baseline: 1.0394x; 1.0394x over previous
import functools

import jax
import jax.numpy as jnp
from jax import lax
from jax.experimental import pallas as pl
from jax.experimental.pallas import tpu as pltpu
from jax.experimental.pallas import tpu_sc as plsc

_INTERP = False

_NC = 2
_NS = 16
_NW = _NC * _NS


def _sc_gather(table, idx_flat):
    V, D = table.shape
    dt = table.dtype
    E = idx_flat.shape[0]
    assert E % _NW == 0 and D % 128 == 0
    b_per_w = E // _NW
    CH = 200
    assert b_per_w % (2 * CH) == 0 and CH % 8 == 0
    n_pairs = b_per_w // (2 * CH)

    mesh = plsc.VectorSubcoreMesh(core_axis_name="c", subcore_axis_name="s",
                                  num_cores=_NC)

    @functools.partial(
        pl.kernel, mesh=mesh,
        out_type=jax.ShapeDtypeStruct((E, D), dt),
        scratch_types=[
            pltpu.VMEM((CH,), jnp.int32),
            pltpu.VMEM((CH,), jnp.int32),
            pltpu.VMEM((CH, D), dt),
            pltpu.VMEM((CH, D), dt),
            pltpu.SemaphoreType.DMA,
            pltpu.SemaphoreType.DMA,
            pltpu.SemaphoreType.DMA,
            pltpu.SemaphoreType.DMA,
        ],
    )
    def k(table_hbm, idx_hbm, out_hbm, idx_a, idx_b, rows_a, rows_b,
          gsem_a, gsem_b, wsem_a, wsem_b):
        wid = lax.axis_index("s") * _NC + lax.axis_index("c")
        base = wid * b_per_w

        def start_gather(off, idx_v, rows_v, gsem):
            pltpu.sync_copy(idx_hbm.at[pl.ds(off, CH)], idx_v)
            pltpu.async_copy(table_hbm.at[idx_v], rows_v, gsem)

        def wait_gather(idx_v, rows_v, gsem):
            pltpu.make_async_copy(table_hbm.at[idx_v], rows_v, gsem).wait()

        def drain_write(rows_v, wsem):
            pltpu.make_async_copy(rows_v, out_hbm.at[pl.ds(base, CH)],
                                  wsem).wait()

        def body(j, carry):
            off0 = base + (2 * j) * CH
            off1 = off0 + CH

            @pl.when(j > 0)
            def _():
                drain_write(rows_a, wsem_a)

            start_gather(off0, idx_a, rows_a, gsem_a)

            @pl.when(j > 0)
            def _():
                drain_write(rows_b, wsem_b)

            start_gather(off1, idx_b, rows_b, gsem_b)

            wait_gather(idx_a, rows_a, gsem_a)
            pltpu.async_copy(rows_a, out_hbm.at[pl.ds(off0, CH)], wsem_a)
            wait_gather(idx_b, rows_b, gsem_b)
            pltpu.async_copy(rows_b, out_hbm.at[pl.ds(off1, CH)], wsem_b)
            return carry

        lax.fori_loop(0, n_pairs, body, 0)
        drain_write(rows_a, wsem_a)
        drain_write(rows_b, wsem_b)

    return k(table, idx_flat)


def _conv_call(x, gath, nbrf, Ws, Wf, PA, PB, Wnext, y_dtype):
    N, AF = x.shape
    E, NF = nbrf.shape
    C = gath.shape[1]
    M = E // N
    A = 200
    T = N // A
    ET = A * M
    cnt1 = float(E)

    def body(x_ref, g_ref, nb_ref, ws_ref, wf_ref, pa_ref, pb_ref, wn_ref,
             xo_ref, yo_ref, ns_ref, st_ref, s2_ref):
        p = pl.program_id(0)
        t = pl.program_id(1)

        @pl.when(jnp.logical_and(p == 0, t == 0))
        def _init():
            st_ref[...] = jnp.zeros_like(st_ref)

        def gated():
            x_t = x_ref[pl.ds(t * A, A), :]
            s2 = jnp.dot(x_t, ws_ref[...],
                         preferred_element_type=jnp.float32) + pa_ref[0:1, :]
            f2 = jnp.dot(nb_ref[...], wf_ref[...],
                         preferred_element_type=jnp.float32)
            g = g_ref[...].astype(jnp.float32)
            z = (g + f2).reshape(A, M, C) + s2[:, None, :]
            return z

        @pl.when(p == 0)
        def _pass0():
            z = gated()
            st_ref[0:1, :] += jnp.sum(z, axis=(0, 1))[None, :]
            st_ref[1:2, :] += jnp.sum(z * z, axis=(0, 1))[None, :]

        @pl.when(jnp.logical_and(p == 1, t == 0))
        def _mkstats():
            m = st_ref[0:1, :] / cnt1
            v = st_ref[1:2, :] / cnt1 - m * m
            sc = pa_ref[1:2, :] * lax.rsqrt(v + 1e-5)
            st_ref[2:3, :] = sc
            st_ref[3:4, :] = pa_ref[2:3, :] - m * sc

        @pl.when(p == 1)
        def _pass1():
            z = gated()
            zn = z * st_ref[2:3, :][:, None, :] + st_ref[3:4, :][:, None, :]
            filt = jax.nn.sigmoid(zn[:, :, :AF])
            core = jax.nn.softplus(zn[:, :, AF:])
            ns_ref[pl.ds(t * A, A), :] = jnp.sum(filt * core, axis=1)

        @pl.when(jnp.logical_and(p == 2, t == 0))
        def _mkstats2():
            ns = ns_ref[...]
            m2 = jnp.mean(ns, axis=0, keepdims=True)
            v2 = jnp.mean((ns - m2) ** 2, axis=0, keepdims=True)
            sc2 = pb_ref[0:1, :] * lax.rsqrt(v2 + 1e-5)
            s2_ref[0:1, :] = sc2
            s2_ref[1:2, :] = pb_ref[1:2, :] - m2 * sc2

        @pl.when(p == 2)
        def _pass2():
            x_t = x_ref[pl.ds(t * A, A), :]
            ns = ns_ref[pl.ds(t * A, A), :]
            xn = jax.nn.softplus(x_t + ns * s2_ref[0:1, :] + s2_ref[1:2, :])
            xo_ref[...] = xn
            yo_ref[...] = jnp.dot(xn, wn_ref[...],
                                  preferred_element_type=jnp.float32
                                  ).astype(yo_ref.dtype)

    grid = (3, T)
    last = T - 1

    def edge_map(p, t):
        return (jnp.where(p == 2, last, t), 0)

    return pl.pallas_call(
        body,
        grid=grid,
        in_specs=[
            pl.BlockSpec((N, AF), lambda p, t: (0, 0)),
            pl.BlockSpec((ET, C), edge_map),
            pl.BlockSpec((ET, NF), edge_map),
            pl.BlockSpec((AF, C), lambda p, t: (0, 0)),
            pl.BlockSpec((NF, C), lambda p, t: (0, 0)),
            pl.BlockSpec((3, C), lambda p, t: (0, 0)),
            pl.BlockSpec((2, AF), lambda p, t: (0, 0)),
            pl.BlockSpec((AF, C), lambda p, t: (0, 0)),
        ],
        out_specs=[
            pl.BlockSpec((A, AF), lambda p, t: (t, 0)),
            pl.BlockSpec((A, C), lambda p, t: (t, 0)),
        ],
        out_shape=[
            jax.ShapeDtypeStruct((N, AF), jnp.float32),
            jax.ShapeDtypeStruct((N, C), y_dtype),
        ],
        scratch_shapes=[
            pltpu.VMEM((N, AF), jnp.float32),
            pltpu.VMEM((4, C), jnp.float32),
            pltpu.VMEM((2, AF), jnp.float32),
        ],
        interpret=_INTERP,
    )(x, gath, nbrf, Ws, Wf, PA, PB, Wnext)


def _emb_call(atom_fea, W_emb, b_emb2, Wn0):
    N, ORIG = atom_fea.shape
    AF = W_emb.shape[1]
    C = Wn0.shape[1]
    R = 1000
    T = N // R

    def body(a_ref, w_ref, b_ref, wn_ref, x_ref, y_ref):
        x = jnp.dot(a_ref[...], w_ref[...],
                    preferred_element_type=jnp.float32) + b_ref[...]
        x_ref[...] = x
        y_ref[...] = jnp.dot(x, wn_ref[...],
                             preferred_element_type=jnp.float32
                             ).astype(y_ref.dtype)

    return pl.pallas_call(
        body,
        grid=(T,),
        in_specs=[
            pl.BlockSpec((R, ORIG), lambda t: (t, 0)),
            pl.BlockSpec((ORIG, AF), lambda t: (0, 0)),
            pl.BlockSpec((1, AF), lambda t: (0, 0)),
            pl.BlockSpec((AF, C), lambda t: (0, 0)),
        ],
        out_specs=[
            pl.BlockSpec((R, AF), lambda t: (t, 0)),
            pl.BlockSpec((R, C), lambda t: (t, 0)),
        ],
        out_shape=[
            jax.ShapeDtypeStruct((N, AF), jnp.float32),
            jax.ShapeDtypeStruct((N, C), jnp.float32),
        ],
        interpret=_INTERP,
    )(atom_fea, W_emb, b_emb2, Wn0)


def _head_call(y3, b_fc2, W_out_row, b_out2, B, PER):
    N, H = y3.shape

    def body(y_ref, bfc_ref, wo_ref, bo_ref, out_ref, cf_ref):
        crys = jnp.mean(y_ref[...].reshape(B, PER, H), axis=1)
        cf = jax.nn.softplus(crys + bfc_ref[...])
        cf_ref[...] = cf
        out_ref[...] = (jnp.sum(cf * wo_ref[...], axis=1, keepdims=True)
                        + bo_ref[...])

    return pl.pallas_call(
        body,
        grid=(1,),
        in_specs=[
            pl.BlockSpec((N, H), lambda i: (0, 0)),
            pl.BlockSpec((1, H), lambda i: (0, 0)),
            pl.BlockSpec((1, H), lambda i: (0, 0)),
            pl.BlockSpec((1, 1), lambda i: (0, 0)),
        ],
        out_specs=[
            pl.BlockSpec((B, 1), lambda i: (0, 0)),
            pl.BlockSpec((B, H), lambda i: (0, 0)),
        ],
        out_shape=[
            jax.ShapeDtypeStruct((B, 1), jnp.float32),
            jax.ShapeDtypeStruct((B, H), jnp.float32),
        ],
        interpret=_INTERP,
    )(y3, b_fc2, W_out_row, b_out2)


def kernel(atom_fea, nbr_fea, nbr_fea_idx, crystal_atom_idx,
           W_emb, b_emb,
           W_full0, b_full0, bn1_g0, bn1_b0, bn2_g0, bn2_b0,
           W_full1, b_full1, bn1_g1, bn1_b1, bn2_g1, bn2_b1,
           W_full2, b_full2, bn1_g2, bn1_b2, bn2_g2, bn2_b2,
           W_fc, b_fc, W_out, b_out):
    N, M, NF = nbr_fea.shape
    AF = W_emb.shape[1]
    B, PER = crystal_atom_idx.shape

    idx_flat = nbr_fea_idx.reshape(N * M).astype(jnp.int32)
    nbrf = nbr_fea.reshape(N * M, NF)

    layers = [
        (W_full0, b_full0, bn1_g0, bn1_b0, bn2_g0, bn2_b0),
        (W_full1, b_full1, bn1_g1, bn1_b1, bn2_g1, bn2_b1),
        (W_full2, b_full2, bn1_g2, bn1_b2, bn2_g2, bn2_b2),
    ]

    x, y = _emb_call(atom_fea, W_emb, b_emb.reshape(1, AF),
                     layers[0][0][AF:2 * AF, :])

    for li, (W_full, b_full, g1, be1, g2, be2) in enumerate(layers):
        Ws = W_full[:AF, :]
        Wf = W_full[2 * AF:, :]
        PA = jnp.stack([b_full, g1, be1], axis=0)
        PB = jnp.stack([g2, be2], axis=0)
        Wnext = layers[li + 1][0][AF:2 * AF, :] if li < 2 else W_fc
        gath = _sc_gather(y, idx_flat)
        x, y = _conv_call(x, gath, nbrf, Ws, Wf, PA, PB, Wnext, jnp.float32)

    out, crys_fea = _head_call(y, b_fc.reshape(1, -1),
                               W_out.reshape(1, -1), b_out.reshape(1, 1),
                               B, PER)
    return (out, crys_fea)
